# baseline (speedup 1.0000x reference)
import functools

import jax
import jax.numpy as jnp
from jax import lax
from jax.experimental import pallas as pl
from jax.experimental.pallas import tpu as pltpu
from jax.experimental.pallas import tpu_sc as plsc

D = 1024
_info = plsc.get_sparse_core_info()
_NC, _NS = _info.num_cores, _info.num_subcores
_NW = _NC * _NS

_G = 16
_NBUF = 4


def _sc_body(seg_hbm, table_hbm, out_hbm, idx_v, buf_all,
             dp0, dp1, dp2, dp3, sem0, sem1, sem2, sem3):
    b = seg_hbm.shape[0]
    b_per_w = b // _NW
    seq = out_hbm.shape[1]
    sid = lax.axis_index("s")
    cid = lax.axis_index("c")
    wid = sid * _NC + cid
    base = wid * b_per_w
    batch = wid // (seq // b_per_w)
    lbase = base - batch * seq
    out_b = out_hbm.at[batch]

    pltpu.sync_copy(seg_hbm.at[pl.ds(base, b_per_w)], idx_v)

    for r in range(16):
        pltpu.sync_copy(table_hbm.at[pl.ds(0, 1)], buf_all.at[pl.ds(r, 1)])
        pltpu.sync_copy(table_hbm.at[pl.ds(1, 1)], buf_all.at[pl.ds(16 + r, 1)])

    dps = (dp0, dp1, dp2, dp3)
    sems = (sem0, sem1, sem2, sem3)
    lanes = lax.iota(jnp.int32, 16)
    n = b_per_w // _G

    descs = [None] * _NBUF

    for g in range(n):
        j = g % _NBUF
        if descs[j] is not None:
            descs[j].wait()
        seg16 = idx_v[pl.ds(g * _G, 16)]
        pos16 = jnp.full((16,), 0, jnp.int32) + (lbase + g * _G) + lanes
        _, perm = plsc.sort_key_val(seg16, pos16)
        dps[j][...] = perm
        k1 = lax.reduce_sum(seg16, axes=(0,))
        descs[j] = pltpu.async_copy(
            buf_all.at[pl.ds(k1, 16)], out_b.at[dps[j]], sems[j])
    for j in range(_NBUF):
        descs[j].wait()


@functools.partial(jax.jit, static_argnums=(2, 3))
def _sc_lookup(seg_flat, table, bsz, seq):
    b = seg_flat.shape[0]
    b_per_w = b // _NW
    mesh = plsc.VectorSubcoreMesh(core_axis_name="c", subcore_axis_name="s")
    return pl.kernel(
        _sc_body,
        out_type=jax.ShapeDtypeStruct((bsz, seq, D), jnp.float32),
        mesh=mesh,
        scratch_types=[
            pltpu.VMEM((b_per_w,), jnp.int32),
            pltpu.VMEM((32, D), jnp.float32),
            pltpu.VMEM((16,), jnp.int32),
            pltpu.VMEM((16,), jnp.int32),
            pltpu.VMEM((16,), jnp.int32),
            pltpu.VMEM((16,), jnp.int32),
            pltpu.SemaphoreType.DMA,
            pltpu.SemaphoreType.DMA,
            pltpu.SemaphoreType.DMA,
            pltpu.SemaphoreType.DMA,
        ],
        compiler_params=pltpu.CompilerParams(
            use_tc_tiling_on_sc=False, needs_layout_passes=False),
    )(seg_flat, table)


def kernel(segments, table):
    bsz, seq = segments.shape
    seg_flat = segments.reshape(bsz * seq).astype(jnp.int32)
    return _sc_lookup(seg_flat, table, bsz, seq)


# speedup vs baseline: 1.0653x; 1.0653x over previous
import functools

import jax
import jax.numpy as jnp
from jax import lax
from jax.experimental import pallas as pl
from jax.experimental.pallas import tpu as pltpu
from jax.experimental.pallas import tpu_sc as plsc

D = 1024
_info = plsc.get_sparse_core_info()
_NC, _NS = _info.num_cores, _info.num_subcores
_NW = _NC * _NS

_CH = 8


def _sc_body(seg_hbm, tab_hbm, out_hbm, idx_v, tab_v, buf0, buf1,
             sem0, sem1):
    b = seg_hbm.shape[0]
    b_per_w = b // _NW
    sid = lax.axis_index("s")
    cid = lax.axis_index("c")
    wid = sid * _NC + cid
    base = wid * b_per_w
    out2 = out_hbm.reshape(b, D)

    pltpu.sync_copy(tab_hbm, tab_v)
    pltpu.sync_copy(seg_hbm.at[pl.ds(base, b_per_w)], idx_v)

    lanes = lax.iota(jnp.int32, 16)
    masks = [lanes == k for k in range(16)]
    zeros = jnp.full((16,), 0, jnp.int32)

    def build(seg16, half, buf):
        for r in range(_CH):
            s = lax.reduce_sum(
                jnp.where(masks[half * _CH + r], seg16, zeros), axes=(0,))
            off = s * D
            for g in range(4):
                vals = [tab_v[pl.ds(off + (g * 16 + j) * 16, 16)]
                        for j in range(16)]
                for j in range(16):
                    buf[r, pl.ds((g * 16 + j) * 16, 16)] = vals[j]

    def scatter(c, buf, sem):
        return pltpu.async_copy(buf, out2.at[pl.ds(base + c * _CH, _CH)], sem)

    n = b_per_w // _CH
    seg16 = idx_v[pl.ds(0, 16)]
    build(seg16, 0, buf0)
    s0 = scatter(0, buf0, sem0)
    build(seg16, 1, buf1)
    s1 = scatter(1, buf1, sem1)

    def pair(t, carry):
        seg16 = idx_v[pl.ds(t * 2 * _CH, 16)]
        s0.wait()
        build(seg16, 0, buf0)
        scatter(2 * t, buf0, sem0)
        s1.wait()
        build(seg16, 1, buf1)
        scatter(2 * t + 1, buf1, sem1)
        return carry

    lax.fori_loop(1, n // 2, pair, jnp.int32(0))
    s0.wait()
    s1.wait()


@functools.partial(jax.jit, static_argnums=(2, 3))
def _sc_lookup(seg_flat, tab_flat, bsz, seq):
    mesh = plsc.VectorSubcoreMesh(core_axis_name="c", subcore_axis_name="s")
    return pl.kernel(
        _sc_body,
        out_type=jax.ShapeDtypeStruct((bsz, seq, D), jnp.float32),
        mesh=mesh,
        scratch_types=[
            pltpu.VMEM((seg_flat.shape[0] // _NW,), jnp.int32),
            pltpu.VMEM((2 * D,), jnp.float32),
            pltpu.VMEM((_CH, D), jnp.float32),
            pltpu.VMEM((_CH, D), jnp.float32),
            pltpu.SemaphoreType.DMA,
            pltpu.SemaphoreType.DMA,
        ],
        compiler_params=pltpu.CompilerParams(
            use_tc_tiling_on_sc=True, needs_layout_passes=False),
    )(seg_flat, tab_flat)


def kernel(segments, table):
    bsz, seq = segments.shape
    seg_flat = segments.reshape(bsz * seq).astype(jnp.int32)
    return _sc_lookup(seg_flat, table.reshape(2 * D), bsz, seq)


# speedup vs baseline: 3.2779x; 3.0770x over previous
import functools

import jax
import jax.numpy as jnp
from jax import lax
from jax.experimental import pallas as pl
from jax.experimental.pallas import tpu as pltpu
from jax.experimental.pallas import tpu_sc as plsc

D = 1024
_info = plsc.get_sparse_core_info()
_NC, _NS = _info.num_cores, _info.num_subcores
_NW = _NC * _NS

_CH = 8


def _sc_body(seg_hbm, tab_hbm, out_hbm, idx_v, tab_v, buf0, buf1,
             sem0, sem1):
    b = seg_hbm.shape[0]
    b_per_w = b // _NW
    sid = lax.axis_index("s")
    cid = lax.axis_index("c")
    wid = sid * _NC + cid
    base = wid * b_per_w
    out2 = out_hbm.reshape(b, D)

    pltpu.sync_copy(tab_hbm, tab_v)
    pltpu.sync_copy(seg_hbm.at[pl.ds(base, b_per_w)], idx_v)

    lanes = lax.iota(jnp.int32, 16)
    masks = [lanes == k for k in range(16)]
    zeros = jnp.full((16,), 0, jnp.int32)

    def build(seg16, half, buf):
        ms = []
        for r in range(_CH):
            s = lax.reduce_sum(
                jnp.where(masks[half * _CH + r], seg16, zeros), axes=(0,))
            ms.append((zeros + s) > 0)
        for g in range(8):
            v0s = [tab_v[pl.ds((g * 8 + j) * 16, 16)] for j in range(8)]
            v1s = [tab_v[pl.ds(D + (g * 8 + j) * 16, 16)] for j in range(8)]
            for j in range(8):
                for r in range(_CH):
                    buf[r, pl.ds((g * 8 + j) * 16, 16)] = jnp.where(
                        ms[r], v1s[j], v0s[j])

    def scatter(c, buf, sem):
        return pltpu.async_copy(buf, out2.at[pl.ds(base + c * _CH, _CH)], sem)

    n = b_per_w // _CH
    seg16 = idx_v[pl.ds(0, 16)]
    build(seg16, 0, buf0)
    s0 = scatter(0, buf0, sem0)
    build(seg16, 1, buf1)
    s1 = scatter(1, buf1, sem1)

    def pair(t, carry):
        seg16 = idx_v[pl.ds(t * 2 * _CH, 16)]
        s0.wait()
        build(seg16, 0, buf0)
        scatter(2 * t, buf0, sem0)
        s1.wait()
        build(seg16, 1, buf1)
        scatter(2 * t + 1, buf1, sem1)
        return carry

    lax.fori_loop(1, n // 2, pair, jnp.int32(0))
    s0.wait()
    s1.wait()


@functools.partial(jax.jit, static_argnums=(2, 3))
def _sc_lookup(seg_flat, tab_flat, bsz, seq):
    mesh = plsc.VectorSubcoreMesh(core_axis_name="c", subcore_axis_name="s")
    return pl.kernel(
        _sc_body,
        out_type=jax.ShapeDtypeStruct((bsz, seq, D), jnp.float32),
        mesh=mesh,
        scratch_types=[
            pltpu.VMEM((seg_flat.shape[0] // _NW,), jnp.int32),
            pltpu.VMEM((2 * D,), jnp.float32),
            pltpu.VMEM((_CH, D), jnp.float32),
            pltpu.VMEM((_CH, D), jnp.float32),
            pltpu.SemaphoreType.DMA,
            pltpu.SemaphoreType.DMA,
        ],
        compiler_params=pltpu.CompilerParams(
            use_tc_tiling_on_sc=True, needs_layout_passes=False),
    )(seg_flat, tab_flat)


def kernel(segments, table):
    bsz, seq = segments.shape
    seg_flat = segments.reshape(bsz * seq).astype(jnp.int32)
    return _sc_lookup(seg_flat, table.reshape(2 * D), bsz, seq)
